# Initial kernel scaffold; baseline (speedup 1.0000x reference)
import jax
import jax.numpy as jnp
from jax.experimental import pallas as pl


def kernel(log_probs, input_lens, labels, den_scores):
    raise NotImplementedError("write your pallas kernel here")


# speedup vs baseline: 25.7526x; 25.7526x over previous
import functools

import numpy as np
import jax
import jax.numpy as jnp
from jax import lax
from jax.experimental import pallas as pl
from jax.experimental.pallas import tpu as pltpu
from jax.experimental.pallas import tpu_sc as plsc

_L = 40
_B, _T, _C = 64, 1024, 43
_NEG = -1e30


def _topology(num_labels):
    s = ["0 0 1", "0 1 2", "1 1 2"]
    for i in range(num_labels):
        sym = 3 + i
        s.append("0 0 %d" % sym)
        s.append("1 0 %d" % sym)
    s.append("0 2 -1")
    s.sort()
    return [tuple(int(x) for x in line.split()) for line in s]


_ARCS = _topology(_L)
_A = len(_ARCS)
_SRCA = np.array([a[0] for a in _ARCS], dtype=np.int32)
_DSTA = np.array([a[1] for a in _ARCS], dtype=np.int32)
_LABA = np.array([a[2] for a in _ARCS], dtype=np.int32)

_PAD = 128
_MS0 = np.zeros((1, _PAD), np.float32)
_MS0[0, :_A] = (_SRCA == 0)
_MS1 = np.zeros((1, _PAD), np.float32)
_MS1[0, :_A] = (_SRCA == 1)
_M0 = np.zeros((_PAD, _PAD), np.float32)
_M1 = np.zeros((_PAD, _PAD), np.float32)
for _a in range(_A):
    if _LABA[_a] >= 0 and _DSTA[_a] == 0:
        if _SRCA[_a] == 0:
            _M0[_a, _LABA[_a]] = 1.0
        else:
            _M1[_a, _LABA[_a]] = 1.0
_OH01 = np.zeros((1, _PAD), np.float32)
_OH11 = np.zeros((1, _PAD), np.float32)
_OHF = np.zeros((1, _PAD), np.float32)
for _a in range(_A):
    if _LABA[_a] < 0:
        _OHF[0, _a] = 1.0
    elif _SRCA[_a] == 0 and _DSTA[_a] == 1:
        _OH01[0, _a] = 1.0
    elif _SRCA[_a] == 1 and _DSTA[_a] == 1:
        _OH11[0, _a] = 1.0

_GB = 8
_GRID = _B // _GB


def _lse2(x, y):
    m = jnp.maximum(x, y)
    return m + jnp.log1p(jnp.exp(-jnp.abs(x - y)))


def _den_body(cs_ref, m_ref, dens_ref, lens_ref, lp_ref, out_ref):
    d = dens_ref[...]
    cs = cs_ref[...]
    ms0 = cs[0:1, :]
    ms1 = cs[1:2, :]
    e = jnp.exp(d)
    lse0 = jnp.log(jnp.sum(e * ms0))
    lse1 = jnp.log(jnp.sum(e * ms1))
    w = d - ms0 * lse0 - ms1 * lse1
    ew = jnp.exp(w) * (ms0 + ms1)
    u0 = jnp.dot(ew, m_ref[0:_PAD, :])
    u1 = jnp.dot(ew, m_ref[_PAD:, :])
    c01 = jnp.sum(w * cs[2:3, :])
    c11 = jnp.sum(w * cs[3:4, :])
    wf = jnp.sum(w * cs[4:5, :])

    lp = lp_ref[...]
    p = jnp.exp(lp)
    u0c = u0[0:1, 0:_C].reshape(1, 1, _C)
    u1c = u1[0:1, 0:_C].reshape(1, 1, _C)
    s00 = jnp.log(jnp.sum(p * u0c, axis=2))
    s10 = jnp.log(jnp.sum(p * u1c, axis=2))
    e2 = lp[:, :, 2]
    s01 = e2 + c01
    s11 = e2 + c11

    lens = lens_ref[...][:, 0:1]
    tt = lax.broadcasted_iota(jnp.int32, (_GB, _T), 1)
    act = tt < lens
    a00 = jnp.where(act, s00, 0.0)
    a01 = jnp.where(act, s01, _NEG)
    a10 = jnp.where(act, s10, _NEG)
    a11 = jnp.where(act, s11, 0.0)

    n = _T
    for k in range(10):
        sh = 1 << k

        def shift(x, fill):
            pad = jnp.full((_GB, sh), fill, x.dtype)
            return jnp.concatenate([pad, x[:, : n - sh]], axis=1)

        b00 = shift(a00, 0.0)
        b01 = shift(a01, _NEG)
        b10 = shift(a10, _NEG)
        b11 = shift(a11, 0.0)
        c00 = _lse2(b00 + a00, b01 + a10)
        c01_ = _lse2(b00 + a01, b01 + a11)
        c10 = _lse2(b10 + a00, b11 + a10)
        c11_ = _lse2(b10 + a01, b11 + a11)
        a00 = jnp.maximum(c00, _NEG)
        a01 = jnp.maximum(c01_, _NEG)
        a10 = jnp.maximum(c10, _NEG)
        a11 = jnp.maximum(c11_, _NEG)

    p00 = a00[:, _T - 1 : _T]
    block_den = jnp.sum(p00) + _GB * wf

    @pl.when(pl.program_id(0) == 0)
    def _():
        out_ref[0, 0] = 0.0

    out_ref[0, 0] += block_den


_CS = np.concatenate([_MS0, _MS1, _OH01, _OH11, _OHF,
                      np.zeros((3, _PAD), np.float32)], axis=0)
_M01 = np.concatenate([_M0, _M1], axis=0)


def _den_call(log_probs, input_lens, den_scores):
    dens = jnp.zeros((1, _PAD), jnp.float32).at[0, :_A].set(den_scores)
    lens2 = jnp.broadcast_to(input_lens[:, None], (_B, 8)).astype(jnp.int32)
    out = pl.pallas_call(
        _den_body,
        grid=(_GRID,),
        in_specs=[
            pl.BlockSpec((8, _PAD), lambda i: (0, 0)),
            pl.BlockSpec((2 * _PAD, _PAD), lambda i: (0, 0)),
            pl.BlockSpec((1, _PAD), lambda i: (0, 0)),
            pl.BlockSpec((_GB, 8), lambda i: (i, 0)),
            pl.BlockSpec((_GB, _T, _C), lambda i: (i, 0, 0)),
        ],
        out_specs=pl.BlockSpec(
            (1, 1), lambda i: (0, 0), memory_space=pltpu.SMEM
        ),
        out_shape=jax.ShapeDtypeStruct((1, 1), jnp.float32),
        compiler_params=pltpu.CompilerParams(
            dimension_semantics=("arbitrary",)
        ),
    )(jnp.asarray(_CS), jnp.asarray(_M01), dens, lens2, log_probs)
    return out[0, 0]


_NW = 32
_BPW = _B // _NW


def _num_body(lp_hbm, lab_hbm, len_hbm, out_hbm, idx_v, gat_v, lab_v, len_v, acc_v, sem):
    wid = lax.axis_index("s") * 2 + lax.axis_index("c")
    pltpu.sync_copy(len_hbm, len_v)
    iot = lax.iota(jnp.int32, 16)
    acc = jnp.zeros((16,), jnp.float32)
    for i in range(_BPW):
        b = wid * _BPW + i
        pltpu.sync_copy(lab_hbm.at[pl.ds(b * _T, _T)], lab_v)
        lenb = plsc.load_gather(len_v, [jnp.full((16,), b, jnp.int32)])
        base = b * (_T * _C)
        for j in range(8):
            for u in range(8):
                t0 = j * 128 + u * 16
                labc = lab_v[pl.ds(t0, 16)]
                idx_v[j, pl.ds(u * 16, 16)] = base + t0 * _C + iot * _C + labc
        copies = [
            pltpu.async_copy(lp_hbm.at[idx_v.at[j]], gat_v.at[j], sem)
            for j in range(8)
        ]
        for c in copies:
            c.wait()
        for j in range(8):
            for u in range(8):
                t0 = j * 128 + u * 16
                v = gat_v[j, pl.ds(u * 16, 16)]
                tvec = t0 + iot
                acc = acc + jnp.where(tvec < lenb, v, 0.0)
    acc_v[...] = acc
    pltpu.sync_copy(acc_v, out_hbm.at[wid])


def _num_call(log_probs, input_lens, labels):
    lp_flat = log_probs.reshape(_B * _T * _C)
    lab_flat = labels.reshape(_B * _T).astype(jnp.int32)
    mesh = plsc.VectorSubcoreMesh(core_axis_name="c", subcore_axis_name="s")
    fn = pl.kernel(
        _num_body,
        out_type=jax.ShapeDtypeStruct((_NW, 16), jnp.float32),
        mesh=mesh,
        scratch_types=[
            pltpu.VMEM((8, 128), jnp.int32),
            pltpu.VMEM((8, 128), jnp.float32),
            pltpu.VMEM((_T,), jnp.int32),
            pltpu.VMEM((_B,), jnp.int32),
            pltpu.VMEM((16,), jnp.float32),
            pltpu.SemaphoreType.DMA,
        ],
        compiler_params=pltpu.CompilerParams(needs_layout_passes=False),
    )
    parts = fn(lp_flat, lab_flat, input_lens.astype(jnp.int32))
    return jnp.sum(parts)


def kernel(log_probs, input_lens, labels, den_scores):
    num = _num_call(log_probs, input_lens, labels)
    den = _den_call(log_probs, input_lens, den_scores)
    return num - den


# speedup vs baseline: 33.9580x; 1.3186x over previous
import functools

import numpy as np
import jax
import jax.numpy as jnp
from jax import lax
from jax.experimental import pallas as pl
from jax.experimental.pallas import tpu as pltpu
from jax.experimental.pallas import tpu_sc as plsc

_L = 40
_B, _T, _C = 64, 1024, 43
_NEG = -1e30


def _topology(num_labels):
    s = ["0 0 1", "0 1 2", "1 1 2"]
    for i in range(num_labels):
        sym = 3 + i
        s.append("0 0 %d" % sym)
        s.append("1 0 %d" % sym)
    s.append("0 2 -1")
    s.sort()
    return [tuple(int(x) for x in line.split()) for line in s]


_ARCS = _topology(_L)
_A = len(_ARCS)
_SRCA = np.array([a[0] for a in _ARCS], dtype=np.int32)
_DSTA = np.array([a[1] for a in _ARCS], dtype=np.int32)
_LABA = np.array([a[2] for a in _ARCS], dtype=np.int32)

_PAD = 128
_MS0 = np.zeros((1, _PAD), np.float32)
_MS0[0, :_A] = (_SRCA == 0)
_MS1 = np.zeros((1, _PAD), np.float32)
_MS1[0, :_A] = (_SRCA == 1)
_M0 = np.zeros((_PAD, _PAD), np.float32)
_M1 = np.zeros((_PAD, _PAD), np.float32)
for _a in range(_A):
    if _LABA[_a] >= 0 and _DSTA[_a] == 0:
        if _SRCA[_a] == 0:
            _M0[_a, _LABA[_a]] = 1.0
        else:
            _M1[_a, _LABA[_a]] = 1.0
_OH01 = np.zeros((1, _PAD), np.float32)
_OH11 = np.zeros((1, _PAD), np.float32)
_OHF = np.zeros((1, _PAD), np.float32)
for _a in range(_A):
    if _LABA[_a] < 0:
        _OHF[0, _a] = 1.0
    elif _SRCA[_a] == 0 and _DSTA[_a] == 1:
        _OH01[0, _a] = 1.0
    elif _SRCA[_a] == 1 and _DSTA[_a] == 1:
        _OH11[0, _a] = 1.0

_GB = 8
_GRID = _B // _GB


def _lse2(x, y):
    m = jnp.maximum(x, y)
    return m + jnp.log1p(jnp.exp(-jnp.abs(x - y)))


def _den_body(cs_ref, m_ref, dens_ref, lens_ref, lp_ref, out_ref):
    d = dens_ref[...]
    cs = cs_ref[...]
    ms0 = cs[0:1, :]
    ms1 = cs[1:2, :]
    e = jnp.exp(d)
    lse0 = jnp.log(jnp.sum(e * ms0))
    lse1 = jnp.log(jnp.sum(e * ms1))
    w = d - ms0 * lse0 - ms1 * lse1
    ew = jnp.exp(w) * (ms0 + ms1)
    u0 = jnp.dot(ew, m_ref[0:_PAD, :])
    u1 = jnp.dot(ew, m_ref[_PAD:, :])
    c01 = jnp.sum(w * cs[2:3, :])
    c11 = jnp.sum(w * cs[3:4, :])
    wf = jnp.sum(w * cs[4:5, :])

    lp = lp_ref[...]
    p = jnp.exp(lp)
    u0c = u0[0:1, 0:_C].reshape(1, 1, _C)
    u1c = u1[0:1, 0:_C].reshape(1, 1, _C)
    s00 = jnp.log(jnp.sum(p * u0c, axis=2))
    s10 = jnp.log(jnp.sum(p * u1c, axis=2))
    e2 = lp[:, :, 2]
    s01 = e2 + c01
    s11 = e2 + c11

    lens = lens_ref[...][:, 0:1]
    tt = lax.broadcasted_iota(jnp.int32, (_GB, _T), 1)
    act = tt < lens
    a00 = jnp.where(act, s00, 0.0)
    a01 = jnp.where(act, s01, _NEG)
    a10 = jnp.where(act, s10, _NEG)
    a11 = jnp.where(act, s11, 0.0)

    n = _T
    for k in range(10):
        sh = 1 << k

        def shift(x, fill):
            pad = jnp.full((_GB, sh), fill, x.dtype)
            return jnp.concatenate([pad, x[:, : n - sh]], axis=1)

        b00 = shift(a00, 0.0)
        b01 = shift(a01, _NEG)
        b10 = shift(a10, _NEG)
        b11 = shift(a11, 0.0)
        c00 = _lse2(b00 + a00, b01 + a10)
        c01_ = _lse2(b00 + a01, b01 + a11)
        c10 = _lse2(b10 + a00, b11 + a10)
        c11_ = _lse2(b10 + a01, b11 + a11)
        a00 = jnp.maximum(c00, _NEG)
        a01 = jnp.maximum(c01_, _NEG)
        a10 = jnp.maximum(c10, _NEG)
        a11 = jnp.maximum(c11_, _NEG)

    p00 = a00[:, _T - 1 : _T]
    block_den = jnp.sum(p00) + _GB * wf

    @pl.when(pl.program_id(0) == 0)
    def _():
        out_ref[0, 0] = 0.0

    out_ref[0, 0] += block_den


_CS = np.concatenate([_MS0, _MS1, _OH01, _OH11, _OHF,
                      np.zeros((3, _PAD), np.float32)], axis=0)
_M01 = np.concatenate([_M0, _M1], axis=0)


def _den_call(log_probs, input_lens, den_scores):
    dens = jnp.zeros((1, _PAD), jnp.float32).at[0, :_A].set(den_scores)
    lens2 = jnp.broadcast_to(input_lens[:, None], (_B, 8)).astype(jnp.int32)
    out = pl.pallas_call(
        _den_body,
        grid=(_GRID,),
        in_specs=[
            pl.BlockSpec((8, _PAD), lambda i: (0, 0)),
            pl.BlockSpec((2 * _PAD, _PAD), lambda i: (0, 0)),
            pl.BlockSpec((1, _PAD), lambda i: (0, 0)),
            pl.BlockSpec((_GB, 8), lambda i: (i, 0)),
            pl.BlockSpec((_GB, _T, _C), lambda i: (i, 0, 0)),
        ],
        out_specs=pl.BlockSpec(
            (1, 1), lambda i: (0, 0), memory_space=pltpu.SMEM
        ),
        out_shape=jax.ShapeDtypeStruct((1, 1), jnp.float32),
        compiler_params=pltpu.CompilerParams(
            dimension_semantics=("arbitrary",)
        ),
    )(jnp.asarray(_CS), jnp.asarray(_M01), dens, lens2, log_probs)
    return out[0, 0]


_NW = 32
_BPW = _B // _NW


def _num_body(lp_hbm, lab_hbm, len_hbm, out_hbm, idx_v, gat_v, lab_v, len_v, acc_v, sem):
    wid = lax.axis_index("s") * 2 + lax.axis_index("c")
    pltpu.sync_copy(len_hbm, len_v)
    iot = lax.iota(jnp.int32, 16)
    acc = jnp.zeros((16,), jnp.float32)
    for i in range(_BPW):
        b = wid * _BPW + i
        pltpu.sync_copy(lab_hbm.at[pl.ds(b * _T, _T)], lab_v)
        lenb = plsc.load_gather(len_v, [jnp.full((16,), b, jnp.int32)])
        base = b * (_T * _C)
        for j in range(8):
            for u in range(8):
                t0 = j * 128 + u * 16
                labc = lab_v[pl.ds(t0, 16)]
                idx_v[j, pl.ds(u * 16, 16)] = base + t0 * _C + iot * _C + labc
        copies = [
            pltpu.async_copy(lp_hbm.at[idx_v.at[j]], gat_v.at[j], sem)
            for j in range(8)
        ]
        for c in copies:
            c.wait()
        for j in range(8):
            for u in range(8):
                t0 = j * 128 + u * 16
                v = gat_v[j, pl.ds(u * 16, 16)]
                tvec = t0 + iot
                acc = acc + jnp.where(tvec < lenb, v, 0.0)
    acc_v[...] = acc
    pltpu.sync_copy(acc_v, out_hbm.at[wid])


def _num_call(log_probs, input_lens, labels):
    lp_flat = log_probs.reshape(_B * _T * _C)
    lab_flat = labels.reshape(_B * _T).astype(jnp.int32)
    mesh = plsc.VectorSubcoreMesh(core_axis_name="c", subcore_axis_name="s")
    fn = pl.kernel(
        _num_body,
        out_type=jax.ShapeDtypeStruct((_NW, 16), jnp.float32),
        mesh=mesh,
        scratch_types=[
            pltpu.VMEM((8, 128), jnp.int32),
            pltpu.VMEM((8, 128), jnp.float32),
            pltpu.VMEM((_T,), jnp.int32),
            pltpu.VMEM((_B,), jnp.int32),
            pltpu.VMEM((16,), jnp.float32),
            pltpu.SemaphoreType.DMA,
        ],
        compiler_params=pltpu.CompilerParams(needs_layout_passes=False),
    )
    parts = fn(lp_flat, lab_flat, input_lens.astype(jnp.int32))
    return jnp.sum(parts)


def _den_body_b(cs_ref, m_ref, dens_ref, lens_ref, labs_ref, lp_ref, out_ref):
    d = dens_ref[...]
    cs = cs_ref[...]
    ms0 = cs[0:1, :]
    ms1 = cs[1:2, :]
    e = jnp.exp(d)
    lse0 = jnp.log(jnp.sum(e * ms0))
    lse1 = jnp.log(jnp.sum(e * ms1))
    w = d - ms0 * lse0 - ms1 * lse1
    ew = jnp.exp(w) * (ms0 + ms1)
    u0 = jnp.dot(ew, m_ref[0:_PAD, :])
    u1 = jnp.dot(ew, m_ref[_PAD:, :])
    c01 = jnp.sum(w * cs[2:3, :])
    c11 = jnp.sum(w * cs[3:4, :])
    wf = jnp.sum(w * cs[4:5, :])

    lp = lp_ref[...]
    p = jnp.exp(lp)
    u0c = u0[0:1, 0:_C].reshape(1, 1, _C)
    u1c = u1[0:1, 0:_C].reshape(1, 1, _C)
    s00 = jnp.log(jnp.sum(p * u0c, axis=2))
    s10 = jnp.log(jnp.sum(p * u1c, axis=2))
    e2 = lp[:, :, 2]
    s01 = e2 + c01
    s11 = e2 + c11

    lens = lens_ref[...][:, 0:1]
    tt = lax.broadcasted_iota(jnp.int32, (_GB, _T), 1)
    act = tt < lens

    labs = labs_ref[...]
    ci = lax.broadcasted_iota(jnp.int32, (_GB, _T, _C), 2)
    g = jnp.sum(jnp.where(ci == labs[:, :, None], lp, 0.0), axis=2)
    numpart = jnp.sum(jnp.where(act, g, 0.0))

    a00 = jnp.where(act, s00, 0.0)
    a01 = jnp.where(act, s01, _NEG)
    a10 = jnp.where(act, s10, _NEG)
    a11 = jnp.where(act, s11, 0.0)

    n = _T
    for k in range(10):
        sh = 1 << k

        def shift(x, fill):
            pad = jnp.full((_GB, sh), fill, x.dtype)
            return jnp.concatenate([pad, x[:, : n - sh]], axis=1)

        b00 = shift(a00, 0.0)
        b01 = shift(a01, _NEG)
        b10 = shift(a10, _NEG)
        b11 = shift(a11, 0.0)
        c00 = _lse2(b00 + a00, b01 + a10)
        c01_ = _lse2(b00 + a01, b01 + a11)
        c10 = _lse2(b10 + a00, b11 + a10)
        c11_ = _lse2(b10 + a01, b11 + a11)
        a00 = jnp.maximum(c00, _NEG)
        a01 = jnp.maximum(c01_, _NEG)
        a10 = jnp.maximum(c10, _NEG)
        a11 = jnp.maximum(c11_, _NEG)

    p00 = a00[:, _T - 1 : _T]
    block_out = numpart - (jnp.sum(p00) + _GB * wf)

    @pl.when(pl.program_id(0) == 0)
    def _():
        out_ref[0, 0] = 0.0

    out_ref[0, 0] += block_out


def _fused_call(log_probs, input_lens, labels, den_scores):
    dens = jnp.zeros((1, _PAD), jnp.float32).at[0, :_A].set(den_scores)
    lens2 = jnp.broadcast_to(input_lens[:, None], (_B, 8)).astype(jnp.int32)
    out = pl.pallas_call(
        _den_body_b,
        grid=(_GRID,),
        in_specs=[
            pl.BlockSpec((8, _PAD), lambda i: (0, 0)),
            pl.BlockSpec((2 * _PAD, _PAD), lambda i: (0, 0)),
            pl.BlockSpec((1, _PAD), lambda i: (0, 0)),
            pl.BlockSpec((_GB, 8), lambda i: (i, 0)),
            pl.BlockSpec((_GB, _T), lambda i: (i, 0)),
            pl.BlockSpec((_GB, _T, _C), lambda i: (i, 0, 0)),
        ],
        out_specs=pl.BlockSpec(
            (1, 1), lambda i: (0, 0), memory_space=pltpu.SMEM
        ),
        out_shape=jax.ShapeDtypeStruct((1, 1), jnp.float32),
        compiler_params=pltpu.CompilerParams(
            dimension_semantics=("arbitrary",)
        ),
    )(jnp.asarray(_CS), jnp.asarray(_M01), dens, lens2,
      labels.astype(jnp.int32), log_probs)
    return out[0, 0]


def kernel(log_probs, input_lens, labels, den_scores):
    return _fused_call(log_probs, input_lens, labels, den_scores)


# speedup vs baseline: 44.7895x; 1.3190x over previous
import functools

import numpy as np
import jax
import jax.numpy as jnp
from jax import lax
from jax.experimental import pallas as pl
from jax.experimental.pallas import tpu as pltpu
from jax.experimental.pallas import tpu_sc as plsc

_L = 40
_B, _T, _C = 64, 1024, 43
_NEG = -1e30


def _topology(num_labels):
    s = ["0 0 1", "0 1 2", "1 1 2"]
    for i in range(num_labels):
        sym = 3 + i
        s.append("0 0 %d" % sym)
        s.append("1 0 %d" % sym)
    s.append("0 2 -1")
    s.sort()
    return [tuple(int(x) for x in line.split()) for line in s]


_ARCS = _topology(_L)
_A = len(_ARCS)
_SRCA = np.array([a[0] for a in _ARCS], dtype=np.int32)
_DSTA = np.array([a[1] for a in _ARCS], dtype=np.int32)
_LABA = np.array([a[2] for a in _ARCS], dtype=np.int32)

_PAD = 128
_MS0 = np.zeros((1, _PAD), np.float32)
_MS0[0, :_A] = (_SRCA == 0)
_MS1 = np.zeros((1, _PAD), np.float32)
_MS1[0, :_A] = (_SRCA == 1)
_M0 = np.zeros((_PAD, _PAD), np.float32)
_M1 = np.zeros((_PAD, _PAD), np.float32)
for _a in range(_A):
    if _LABA[_a] >= 0 and _DSTA[_a] == 0:
        if _SRCA[_a] == 0:
            _M0[_a, _LABA[_a]] = 1.0
        else:
            _M1[_a, _LABA[_a]] = 1.0
_OH01 = np.zeros((1, _PAD), np.float32)
_OH11 = np.zeros((1, _PAD), np.float32)
_OHF = np.zeros((1, _PAD), np.float32)
for _a in range(_A):
    if _LABA[_a] < 0:
        _OHF[0, _a] = 1.0
    elif _SRCA[_a] == 0 and _DSTA[_a] == 1:
        _OH01[0, _a] = 1.0
    elif _SRCA[_a] == 1 and _DSTA[_a] == 1:
        _OH11[0, _a] = 1.0
_OHC2 = np.zeros((1, _PAD), np.float32)
_OHC2[0, 2] = 1.0

_CS = np.concatenate([_MS0, _MS1, _OH01, _OH11, _OHF, _OHC2,
                      np.zeros((2, _PAD), np.float32)], axis=0)
_M01 = np.concatenate([_M0, _M1], axis=0)

_GB = 8
_GRID = _B // _GB


def _lse2(x, y):
    m = jnp.maximum(x, y)
    return m + jnp.log1p(jnp.exp(-jnp.abs(x - y)))


def _den_body(cs_ref, m_ref, dens_ref, lens_ref, lp_ref, out_ref):
    d = dens_ref[...]
    cs = cs_ref[...]
    ms0 = cs[0:1, :]
    ms1 = cs[1:2, :]
    e = jnp.exp(d)
    lse0 = jnp.log(jnp.sum(e * ms0))
    lse1 = jnp.log(jnp.sum(e * ms1))
    w = d - ms0 * lse0 - ms1 * lse1
    ew = jnp.exp(w) * (ms0 + ms1)
    u0 = jnp.dot(ew, m_ref[0:_PAD, :])
    u1 = jnp.dot(ew, m_ref[_PAD:, :])
    c01 = jnp.sum(w * cs[2:3, :])
    c11 = jnp.sum(w * cs[3:4, :])
    wf = jnp.sum(w * cs[4:5, :])

    lp = lp_ref[...]
    pm = jnp.exp(lp).reshape(_GB * _T, _C)
    ustack = jnp.concatenate([u0, u1, cs[5:6, :]], axis=0)[:, 0:_C]
    ucols = jnp.transpose(ustack)
    v = jax.lax.dot_general(
        pm, ucols, (((1,), (0,)), ((), ())),
        preferred_element_type=jnp.float32,
    )
    v3 = v.reshape(_GB, _T, 3)
    vt = jnp.transpose(v3, (0, 2, 1))
    s00 = jnp.log(vt[:, 0, :])
    s10 = jnp.log(vt[:, 1, :])
    e2 = jnp.log(vt[:, 2, :])
    s01 = e2 + c01
    s11 = e2 + c11

    lens = lens_ref[...][:, 0:1]
    tt = lax.broadcasted_iota(jnp.int32, (_GB, _T), 1)
    act = tt < lens
    a00 = jnp.where(act, s00, 0.0)
    a01 = jnp.where(act, s01, _NEG)
    a10 = jnp.where(act, s10, _NEG)
    a11 = jnp.where(act, s11, 0.0)

    n = _T
    for k in range(10):
        sh = 1 << k

        def shift(x, fill):
            pad = jnp.full((_GB, sh), fill, x.dtype)
            return jnp.concatenate([pad, x[:, : n - sh]], axis=1)

        b00 = shift(a00, 0.0)
        b01 = shift(a01, _NEG)
        b10 = shift(a10, _NEG)
        b11 = shift(a11, 0.0)
        c00 = _lse2(b00 + a00, b01 + a10)
        a00_new = jnp.maximum(c00, _NEG)
        if k < 9:
            c01_ = _lse2(b00 + a01, b01 + a11)
            c10 = _lse2(b10 + a00, b11 + a10)
            c11_ = _lse2(b10 + a01, b11 + a11)
            a01 = jnp.maximum(c01_, _NEG)
            a10 = jnp.maximum(c10, _NEG)
            a11 = jnp.maximum(c11_, _NEG)
        a00 = a00_new

    p00 = a00[:, _T - 1 : _T]
    block_den = jnp.sum(p00) + _GB * wf

    @pl.when(pl.program_id(0) == 0)
    def _():
        out_ref[0, 0] = 0.0

    out_ref[0, 0] += block_den


def _den_call(log_probs, input_lens, den_scores):
    dens = jnp.zeros((1, _PAD), jnp.float32).at[0, :_A].set(den_scores)
    lens2 = jnp.broadcast_to(input_lens[:, None], (_B, 8)).astype(jnp.int32)
    out = pl.pallas_call(
        _den_body,
        grid=(_GRID,),
        in_specs=[
            pl.BlockSpec((8, _PAD), lambda i: (0, 0)),
            pl.BlockSpec((2 * _PAD, _PAD), lambda i: (0, 0)),
            pl.BlockSpec((1, _PAD), lambda i: (0, 0)),
            pl.BlockSpec((_GB, 8), lambda i: (i, 0)),
            pl.BlockSpec((_GB, _T, _C), lambda i: (i, 0, 0)),
        ],
        out_specs=pl.BlockSpec(
            (1, 1), lambda i: (0, 0), memory_space=pltpu.SMEM
        ),
        out_shape=jax.ShapeDtypeStruct((1, 1), jnp.float32),
        compiler_params=pltpu.CompilerParams(
            dimension_semantics=("arbitrary",)
        ),
    )(jnp.asarray(_CS), jnp.asarray(_M01), dens, lens2, log_probs)
    return out[0, 0]


_NW = 32
_BPW = _B // _NW
_NCH = 8


_CH = 256


def _num_body(lp_hbm, lab_hbm, len_hbm, out_hbm,
              rows_v, lab_v, len_v, acc_v, sem0, sem1):
    wid = lax.axis_index("s") * 2 + lax.axis_index("c")
    sems = [sem0, sem1]
    pltpu.sync_copy(len_hbm, len_v)
    iot = lax.iota(jnp.int32, 16)
    b0 = wid * _BPW
    nchk = _T // _CH
    seq = [(i, c) for i in range(_BPW) for c in range(nchk)]

    def fire(g):
        i, c = seq[g]
        row0 = (b0 + i) * _T + c * _CH
        return pltpu.async_copy(
            lp_hbm.at[pl.ds(row0, _CH)], rows_v.at[g % 2], sems[g % 2]
        )

    cp = fire(0)
    acc = jnp.zeros((16,), jnp.float32)
    lenb = jnp.zeros((16,), jnp.int32)
    for g in range(len(seq)):
        i, c = seq[g]
        b = b0 + i
        if c == 0:
            pltpu.sync_copy(lab_hbm.at[pl.ds(b * _T, _T)], lab_v)
            lenb = plsc.load_gather(len_v, [jnp.full((16,), b, jnp.int32)])
        nxt = fire(g + 1) if g + 1 < len(seq) else None
        cp.wait()
        bufv = jnp.full((16,), g % 2, jnp.int32)
        for u in range(_CH // 16):
            rloc = u * 16 + iot
            t0 = c * _CH + u * 16
            labc = lab_v[pl.ds(t0, 16)]
            val = plsc.load_gather(rows_v, [bufv, rloc, labc])
            acc = acc + jnp.where(t0 + iot < lenb, val, 0.0)
        cp = nxt
    acc_v[...] = acc
    pltpu.sync_copy(acc_v, out_hbm.at[wid])


def _num_call(log_probs, input_lens, labels):
    lp_rows = log_probs.reshape(_B * _T, _C)
    lab_flat = labels.reshape(_B * _T).astype(jnp.int32)
    mesh = plsc.VectorSubcoreMesh(core_axis_name="c", subcore_axis_name="s")
    fn = pl.kernel(
        _num_body,
        out_type=jax.ShapeDtypeStruct((_NW, 16), jnp.float32),
        mesh=mesh,
        scratch_types=[
            pltpu.VMEM((2, _CH, _C), jnp.float32),
            pltpu.VMEM((_T,), jnp.int32),
            pltpu.VMEM((_B,), jnp.int32),
            pltpu.VMEM((16,), jnp.float32),
            pltpu.SemaphoreType.DMA,
            pltpu.SemaphoreType.DMA,
        ],
        compiler_params=pltpu.CompilerParams(needs_layout_passes=False),
    )
    parts = fn(lp_rows, lab_flat, input_lens.astype(jnp.int32))
    return jnp.sum(parts)


def kernel(log_probs, input_lens, labels, den_scores):
    num = _num_call(log_probs, input_lens, labels)
    den = _den_call(log_probs, input_lens, den_scores)
    return num - den


# speedup vs baseline: 45.3006x; 1.0114x over previous
import functools

import numpy as np
import jax
import jax.numpy as jnp
from jax import lax
from jax.experimental import pallas as pl
from jax.experimental.pallas import tpu as pltpu
from jax.experimental.pallas import tpu_sc as plsc

_L = 40
_B, _T, _C = 64, 1024, 43
_NEG = -1e30


def _topology(num_labels):
    s = ["0 0 1", "0 1 2", "1 1 2"]
    for i in range(num_labels):
        sym = 3 + i
        s.append("0 0 %d" % sym)
        s.append("1 0 %d" % sym)
    s.append("0 2 -1")
    s.sort()
    return [tuple(int(x) for x in line.split()) for line in s]


_ARCS = _topology(_L)
_A = len(_ARCS)
_SRCA = np.array([a[0] for a in _ARCS], dtype=np.int32)
_DSTA = np.array([a[1] for a in _ARCS], dtype=np.int32)
_LABA = np.array([a[2] for a in _ARCS], dtype=np.int32)

_PAD = 128
_MS0 = np.zeros((1, _PAD), np.float32)
_MS0[0, :_A] = (_SRCA == 0)
_MS1 = np.zeros((1, _PAD), np.float32)
_MS1[0, :_A] = (_SRCA == 1)
_M0 = np.zeros((_PAD, _PAD), np.float32)
_M1 = np.zeros((_PAD, _PAD), np.float32)
for _a in range(_A):
    if _LABA[_a] >= 0 and _DSTA[_a] == 0:
        if _SRCA[_a] == 0:
            _M0[_a, _LABA[_a]] = 1.0
        else:
            _M1[_a, _LABA[_a]] = 1.0
_OH01 = np.zeros((1, _PAD), np.float32)
_OH11 = np.zeros((1, _PAD), np.float32)
_OHF = np.zeros((1, _PAD), np.float32)
for _a in range(_A):
    if _LABA[_a] < 0:
        _OHF[0, _a] = 1.0
    elif _SRCA[_a] == 0 and _DSTA[_a] == 1:
        _OH01[0, _a] = 1.0
    elif _SRCA[_a] == 1 and _DSTA[_a] == 1:
        _OH11[0, _a] = 1.0
_OHC2 = np.zeros((1, _PAD), np.float32)
_OHC2[0, 2] = 1.0

_CS = np.concatenate([_MS0, _MS1, _OH01, _OH11, _OHF, _OHC2,
                      np.zeros((2, _PAD), np.float32)], axis=0)
_M01 = np.concatenate([_M0, _M1], axis=0)

_GB = 8
_GRID = _B // _GB


def _lse2(x, y):
    m = jnp.maximum(x, y)
    return m + jnp.log1p(jnp.exp(-jnp.abs(x - y)))


def _den_body(cs_ref, m_ref, dens_ref, lens_ref, lp_ref, out_ref):
    d = dens_ref[...]
    cs = cs_ref[...]
    ms0 = cs[0:1, :]
    ms1 = cs[1:2, :]
    e = jnp.exp(d)
    lse0 = jnp.log(jnp.sum(e * ms0))
    lse1 = jnp.log(jnp.sum(e * ms1))
    w = d - ms0 * lse0 - ms1 * lse1
    ew = jnp.exp(w) * (ms0 + ms1)
    u0 = jnp.dot(ew, m_ref[0:_PAD, :])
    u1 = jnp.dot(ew, m_ref[_PAD:, :])
    c01 = jnp.sum(w * cs[2:3, :])
    c11 = jnp.sum(w * cs[3:4, :])
    wf = jnp.sum(w * cs[4:5, :])

    lp = lp_ref[...]
    pm = jnp.exp(lp).reshape(_GB * _T, _C)
    ustack = jnp.concatenate([u0, u1, cs[5:6, :]], axis=0)[:, 0:_C]
    ucols = jnp.transpose(ustack)
    v = jax.lax.dot_general(
        pm, ucols, (((1,), (0,)), ((), ())),
        preferred_element_type=jnp.float32,
    )
    v3 = v.reshape(_GB, _T, 3)
    vt = jnp.transpose(v3, (0, 2, 1))
    s00 = jnp.log(vt[:, 0, :])
    s10 = jnp.log(vt[:, 1, :])
    e2 = jnp.log(vt[:, 2, :])
    s01 = e2 + c01
    s11 = e2 + c11

    lens = lens_ref[...][:, 0:1]
    tt = lax.broadcasted_iota(jnp.int32, (_GB, _T), 1)
    act = tt < lens
    a00 = jnp.where(act, s00, 0.0)
    a01 = jnp.where(act, s01, _NEG)
    a10 = jnp.where(act, s10, _NEG)
    a11 = jnp.where(act, s11, 0.0)

    n = _T
    for k in range(10):
        sh = 1 << k

        def shift(x, fill):
            pad = jnp.full((_GB, sh), fill, x.dtype)
            return jnp.concatenate([pad, x[:, : n - sh]], axis=1)

        b00 = shift(a00, 0.0)
        b01 = shift(a01, _NEG)
        b10 = shift(a10, _NEG)
        b11 = shift(a11, 0.0)
        c00 = _lse2(b00 + a00, b01 + a10)
        a00_new = jnp.maximum(c00, _NEG)
        if k < 9:
            c01_ = _lse2(b00 + a01, b01 + a11)
            c10 = _lse2(b10 + a00, b11 + a10)
            c11_ = _lse2(b10 + a01, b11 + a11)
            a01 = jnp.maximum(c01_, _NEG)
            a10 = jnp.maximum(c10, _NEG)
            a11 = jnp.maximum(c11_, _NEG)
        a00 = a00_new

    p00 = a00[:, _T - 1 : _T]
    block_den = jnp.sum(p00) + _GB * wf

    @pl.when(pl.program_id(0) == 0)
    def _():
        out_ref[0, 0] = 0.0

    out_ref[0, 0] += block_den


def _den_call(log_probs, input_lens, den_scores):
    dens = jnp.zeros((1, _PAD), jnp.float32).at[0, :_A].set(den_scores)
    lens2 = jnp.broadcast_to(input_lens[:, None], (_B, 8)).astype(jnp.int32)
    out = pl.pallas_call(
        _den_body,
        grid=(_GRID,),
        in_specs=[
            pl.BlockSpec((8, _PAD), lambda i: (0, 0)),
            pl.BlockSpec((2 * _PAD, _PAD), lambda i: (0, 0)),
            pl.BlockSpec((1, _PAD), lambda i: (0, 0)),
            pl.BlockSpec((_GB, 8), lambda i: (i, 0)),
            pl.BlockSpec((_GB, _T, _C), lambda i: (i, 0, 0)),
        ],
        out_specs=pl.BlockSpec(
            (1, 1), lambda i: (0, 0), memory_space=pltpu.SMEM
        ),
        out_shape=jax.ShapeDtypeStruct((1, 1), jnp.float32),
        compiler_params=pltpu.CompilerParams(
            dimension_semantics=("arbitrary",)
        ),
    )(jnp.asarray(_CS), jnp.asarray(_M01), dens, lens2, log_probs)
    return out[0, 0]


_NW = 32
_BPW = _B // _NW
_NCH = 8


_CH = 256


def _num_body(lp_hbm, lab_hbm, len_hbm, out_hbm,
              rows_v, lab_v, len_v, acc_v, sem0, sem1):
    wid = lax.axis_index("s") * 2 + lax.axis_index("c")
    sems = [sem0, sem1]
    pltpu.sync_copy(len_hbm, len_v)
    iot = lax.iota(jnp.int32, 16)
    b0 = wid * _BPW
    nchk = _T // _CH
    seq = [(i, c) for i in range(_BPW) for c in range(nchk)]

    def fire(g):
        i, c = seq[g]
        return pltpu.async_copy(
            lp_hbm.at[b0 + i, pl.ds(c * _CH, _CH)], rows_v.at[g % 2],
            sems[g % 2]
        )

    cp = fire(0)
    acc = jnp.zeros((16,), jnp.float32)
    lenb = jnp.zeros((16,), jnp.int32)
    for g in range(len(seq)):
        i, c = seq[g]
        b = b0 + i
        if c == 0:
            pltpu.sync_copy(lab_hbm.at[b], lab_v)
            lenb = plsc.load_gather(len_v, [jnp.full((16,), b, jnp.int32)])
        nxt = fire(g + 1) if g + 1 < len(seq) else None
        cp.wait()
        bufv = jnp.full((16,), g % 2, jnp.int32)
        for u in range(_CH // 16):
            rloc = u * 16 + iot
            t0 = c * _CH + u * 16
            labc = lab_v[pl.ds(t0, 16)]
            val = plsc.load_gather(rows_v, [bufv, rloc, labc])
            acc = acc + jnp.where(t0 + iot < lenb, val, 0.0)
        cp = nxt
    acc_v[...] = acc
    pltpu.sync_copy(acc_v, out_hbm.at[wid])


def _num_call(log_probs, input_lens, labels):
    lp_rows = log_probs
    lab_flat = labels.astype(jnp.int32)
    mesh = plsc.VectorSubcoreMesh(core_axis_name="c", subcore_axis_name="s")
    fn = pl.kernel(
        _num_body,
        out_type=jax.ShapeDtypeStruct((_NW, 16), jnp.float32),
        mesh=mesh,
        scratch_types=[
            pltpu.VMEM((2, _CH, _C), jnp.float32),
            pltpu.VMEM((_T,), jnp.int32),
            pltpu.VMEM((_B,), jnp.int32),
            pltpu.VMEM((16,), jnp.float32),
            pltpu.SemaphoreType.DMA,
            pltpu.SemaphoreType.DMA,
        ],
        compiler_params=pltpu.CompilerParams(needs_layout_passes=False),
    )
    parts = fn(lp_rows, lab_flat, input_lens.astype(jnp.int32))
    return jnp.sum(parts)


def kernel(log_probs, input_lens, labels, den_scores):
    num = _num_call(log_probs, input_lens, labels)
    den = _den_call(log_probs, input_lens, den_scores)
    return num - den


# speedup vs baseline: 45.3548x; 1.0012x over previous
import functools

import numpy as np
import jax
import jax.numpy as jnp
from jax import lax
from jax.experimental import pallas as pl
from jax.experimental.pallas import tpu as pltpu
from jax.experimental.pallas import tpu_sc as plsc

_L = 40
_B, _T, _C = 64, 1024, 43
_NEG = -1e30


def _topology(num_labels):
    s = ["0 0 1", "0 1 2", "1 1 2"]
    for i in range(num_labels):
        sym = 3 + i
        s.append("0 0 %d" % sym)
        s.append("1 0 %d" % sym)
    s.append("0 2 -1")
    s.sort()
    return [tuple(int(x) for x in line.split()) for line in s]


_ARCS = _topology(_L)
_A = len(_ARCS)
_SRCA = np.array([a[0] for a in _ARCS], dtype=np.int32)
_DSTA = np.array([a[1] for a in _ARCS], dtype=np.int32)
_LABA = np.array([a[2] for a in _ARCS], dtype=np.int32)

_PAD = 128
_MS0 = np.zeros((1, _PAD), np.float32)
_MS0[0, :_A] = (_SRCA == 0)
_MS1 = np.zeros((1, _PAD), np.float32)
_MS1[0, :_A] = (_SRCA == 1)
_M0 = np.zeros((_PAD, _PAD), np.float32)
_M1 = np.zeros((_PAD, _PAD), np.float32)
for _a in range(_A):
    if _LABA[_a] >= 0 and _DSTA[_a] == 0:
        if _SRCA[_a] == 0:
            _M0[_a, _LABA[_a]] = 1.0
        else:
            _M1[_a, _LABA[_a]] = 1.0
_OH01 = np.zeros((1, _PAD), np.float32)
_OH11 = np.zeros((1, _PAD), np.float32)
_OHF = np.zeros((1, _PAD), np.float32)
for _a in range(_A):
    if _LABA[_a] < 0:
        _OHF[0, _a] = 1.0
    elif _SRCA[_a] == 0 and _DSTA[_a] == 1:
        _OH01[0, _a] = 1.0
    elif _SRCA[_a] == 1 and _DSTA[_a] == 1:
        _OH11[0, _a] = 1.0
_OHC2 = np.zeros((1, _PAD), np.float32)
_OHC2[0, 2] = 1.0

_CS = np.concatenate([_MS0, _MS1, _OH01, _OH11, _OHF, _OHC2,
                      np.zeros((2, _PAD), np.float32)], axis=0)
_M01 = np.concatenate([_M0, _M1], axis=0)

_GB = 8
_GRID = _B // _GB


def _lse2(x, y):
    m = jnp.maximum(x, y)
    return m + jnp.log1p(jnp.exp(-jnp.abs(x - y)))


def _den_body(cs_ref, m_ref, dens_ref, lens_ref, lp_ref, out_ref):
    d = dens_ref[...]
    cs = cs_ref[...]
    ms0 = cs[0:1, :]
    ms1 = cs[1:2, :]
    e = jnp.exp(d)
    lse0 = jnp.log(jnp.sum(e * ms0))
    lse1 = jnp.log(jnp.sum(e * ms1))
    w = d - ms0 * lse0 - ms1 * lse1
    ew = jnp.exp(w) * (ms0 + ms1)
    u0 = jnp.dot(ew, m_ref[0:_PAD, :])
    u1 = jnp.dot(ew, m_ref[_PAD:, :])
    c01 = jnp.sum(w * cs[2:3, :])
    c11 = jnp.sum(w * cs[3:4, :])
    wf = jnp.sum(w * cs[4:5, :])

    lp = lp_ref[...]
    pm = jnp.exp(lp).reshape(_GB * _T, _C)
    ustack = jnp.concatenate([u0, u1, cs[5:6, :]], axis=0)[:, 0:_C]
    ucols = jnp.transpose(ustack)
    v = jax.lax.dot_general(
        pm, ucols, (((1,), (0,)), ((), ())),
        preferred_element_type=jnp.float32,
    )
    v3 = v.reshape(_GB, _T, 3)
    vt = jnp.transpose(v3, (0, 2, 1))
    s00 = jnp.log(vt[:, 0, :])
    s10 = jnp.log(vt[:, 1, :])
    e2 = jnp.log(vt[:, 2, :])
    s01 = e2 + c01
    s11 = e2 + c11

    lens = lens_ref[...][:, 0:1]
    tt = lax.broadcasted_iota(jnp.int32, (_GB, _T), 1)
    act = tt < lens
    a00 = jnp.where(act, s00, 0.0)
    a01 = jnp.where(act, s01, _NEG)
    a10 = jnp.where(act, s10, _NEG)
    a11 = jnp.where(act, s11, 0.0)

    n = _T
    for k in range(10):
        sh = 1 << k

        def shift(x, fill):
            pad = jnp.full((_GB, sh), fill, x.dtype)
            return jnp.concatenate([pad, x[:, : n - sh]], axis=1)

        b00 = shift(a00, 0.0)
        b01 = shift(a01, _NEG)
        b10 = shift(a10, _NEG)
        b11 = shift(a11, 0.0)
        c00 = _lse2(b00 + a00, b01 + a10)
        a00_new = jnp.maximum(c00, _NEG)
        if k < 9:
            c01_ = _lse2(b00 + a01, b01 + a11)
            c10 = _lse2(b10 + a00, b11 + a10)
            c11_ = _lse2(b10 + a01, b11 + a11)
            a01 = jnp.maximum(c01_, _NEG)
            a10 = jnp.maximum(c10, _NEG)
            a11 = jnp.maximum(c11_, _NEG)
        a00 = a00_new

    p00 = a00[:, _T - 1 : _T]
    block_den = jnp.sum(p00) + _GB * wf

    @pl.when(pl.program_id(0) == 0)
    def _():
        out_ref[0, 0] = 0.0

    out_ref[0, 0] += block_den


def _den_call(log_probs, input_lens, den_scores):
    dens = jnp.zeros((1, _PAD), jnp.float32).at[0, :_A].set(den_scores)
    lens2 = jnp.broadcast_to(input_lens[:, None], (_B, 8)).astype(jnp.int32)
    out = pl.pallas_call(
        _den_body,
        grid=(_GRID,),
        in_specs=[
            pl.BlockSpec((8, _PAD), lambda i: (0, 0)),
            pl.BlockSpec((2 * _PAD, _PAD), lambda i: (0, 0)),
            pl.BlockSpec((1, _PAD), lambda i: (0, 0)),
            pl.BlockSpec((_GB, 8), lambda i: (i, 0)),
            pl.BlockSpec((_GB, _T, _C), lambda i: (i, 0, 0)),
        ],
        out_specs=pl.BlockSpec(
            (1, 1), lambda i: (0, 0), memory_space=pltpu.SMEM
        ),
        out_shape=jax.ShapeDtypeStruct((1, 1), jnp.float32),
        compiler_params=pltpu.CompilerParams(
            dimension_semantics=("arbitrary",)
        ),
    )(jnp.asarray(_CS), jnp.asarray(_M01), dens, lens2, log_probs)
    return out[0, 0]


_NW = 32
_BPW = _B // _NW
_NCH = 8


_CH = 256


def _num_body(lp_hbm, lab_hbm, len_hbm, out_hbm,
              rows_v, lab_v, len_v, acc_v, sem0, sem1):
    wid = lax.axis_index("s") * 2 + lax.axis_index("c")
    sems = [sem0, sem1]
    pltpu.sync_copy(len_hbm, len_v)
    iot = lax.iota(jnp.int32, 16)
    b0 = wid * _BPW
    nchk = _T // _CH
    seq = [(i, c) for i in range(_BPW) for c in range(nchk)]

    def fire(g):
        i, c = seq[g]
        return pltpu.async_copy(
            lp_hbm.at[b0 + i, pl.ds(c * _CH, _CH)], rows_v.at[g % 2],
            sems[g % 2]
        )

    cp = fire(0)
    acc = jnp.zeros((16,), jnp.float32)
    lenb = jnp.zeros((16,), jnp.int32)
    for g in range(len(seq)):
        i, c = seq[g]
        b = b0 + i
        if c == 0:
            pltpu.sync_copy(lab_hbm.at[b], lab_v)
            lenb = plsc.load_gather(len_v, [jnp.full((16,), b, jnp.int32)])
        nxt = fire(g + 1) if g + 1 < len(seq) else None
        cp.wait()
        bufv = jnp.full((16,), g % 2, jnp.int32)
        for u in range(_CH // 16):
            rloc = u * 16 + iot
            t0 = c * _CH + u * 16
            labc = lab_v[pl.ds(t0, 16)]
            val = plsc.load_gather(rows_v, [bufv, rloc, labc])
            acc = acc + jnp.where(t0 + iot < lenb, val, 0.0)
        cp = nxt
    acc_v[...] = acc
    pltpu.sync_copy(acc_v, out_hbm.at[wid])


def _num_call(log_probs, input_lens, labels):
    lp_rows = log_probs
    lab_flat = labels.astype(jnp.int32)
    mesh = plsc.VectorSubcoreMesh(core_axis_name="c", subcore_axis_name="s")
    fn = pl.kernel(
        _num_body,
        out_type=jax.ShapeDtypeStruct((_NW, 16), jnp.float32),
        mesh=mesh,
        scratch_types=[
            pltpu.VMEM((2, _CH, _C), jnp.float32),
            pltpu.VMEM((_T,), jnp.int32),
            pltpu.VMEM((_B,), jnp.int32),
            pltpu.VMEM((16,), jnp.float32),
            pltpu.SemaphoreType.DMA,
            pltpu.SemaphoreType.DMA,
        ],
        compiler_params=pltpu.CompilerParams(needs_layout_passes=False,
                                             use_tc_tiling_on_sc=True),
    )
    parts = fn(lp_rows, lab_flat, input_lens.astype(jnp.int32))
    return jnp.sum(parts)


def kernel(log_probs, input_lens, labels, den_scores):
    num = _num_call(log_probs, input_lens, labels)
    den = _den_call(log_probs, input_lens, den_scores)
    return num - den
